# baseline (speedup 1.0000x reference)
import functools

import jax
import jax.numpy as jnp
from jax import lax
from jax.experimental import pallas as pl
from jax.experimental.pallas import tpu as pltpu
from jax.experimental.pallas import tpu_sc as plsc

NC = 2
NS = 16
NW = NC * NS
CHUNK = 128


def _cdiv(a, b):
    return (a + b - 1) // b


DW = 8


@functools.partial(jax.jit, static_argnames=("n_pad", "nch"))
def _sc_degree(dst3, ones_c, zeros_d, *, n_pad, nch):
    zrows = n_pad // NS
    mesh = plsc.VectorSubcoreMesh(core_axis_name="c", subcore_axis_name="s")

    @functools.partial(
        pl.kernel,
        out_type=jax.ShapeDtypeStruct((NC, n_pad, DW), jnp.float32),
        mesh=mesh,
        scratch_types=[
            pltpu.VMEM((nch, CHUNK), jnp.int32),
            pltpu.VMEM((CHUNK, DW), jnp.float32),
            pltpu.VMEM((zrows, DW), jnp.float32),
            pltpu.VMEM_SHARED((n_pad, DW), jnp.float32),
        ],
        compiler_params=pltpu.CompilerParams(use_tc_tiling_on_sc=False),
    )
    def deg_kernel(dst_hbm, ones_hbm, zero_hbm, out_hbm,
                   dst_v, ones_v, stage_v, acc_sh):
        c = lax.axis_index("c")
        s = lax.axis_index("s")
        wid = s * NC + c
        pltpu.sync_copy(dst_hbm.at[wid], dst_v)
        pltpu.sync_copy(ones_hbm, ones_v)
        r0 = s * zrows
        pltpu.sync_copy(zero_hbm.at[pl.ds(r0, zrows)], stage_v)
        pltpu.sync_copy(stage_v, acc_sh.at[pl.ds(r0, zrows)])
        plsc.subcore_barrier()

        def body(j, carry):
            pltpu.sync_copy(ones_v, acc_sh.at[dst_v.at[j]], add=True)
            return carry

        lax.fori_loop(0, nch, body, 0)
        plsc.subcore_barrier()
        pltpu.sync_copy(acc_sh.at[pl.ds(r0, zrows)], stage_v)
        pltpu.sync_copy(stage_v, out_hbm.at[c, pl.ds(r0, zrows)])

    return deg_kernel(dst3, ones_c, zeros_d)


NB = 4


@functools.partial(jax.jit, static_argnames=("n_pad", "nch", "hp"))
def _sc_aggregate(y, src3, dst3, zeros_a, *, n_pad, nch, hp):
    zrows = n_pad // NS
    mesh = plsc.VectorSubcoreMesh(core_axis_name="c", subcore_axis_name="s")

    @functools.partial(
        pl.kernel,
        out_type=jax.ShapeDtypeStruct((NC, n_pad, hp), jnp.float32),
        mesh=mesh,
        scratch_types=[
            pltpu.VMEM((nch, CHUNK), jnp.int32),
            pltpu.VMEM((nch, CHUNK), jnp.int32),
            pltpu.VMEM((zrows, hp), jnp.float32),
            pltpu.VMEM_SHARED((n_pad, hp), jnp.float32),
        ]
        + [pltpu.VMEM((CHUNK, hp), jnp.float32) for _ in range(NB)]
        + [pltpu.SemaphoreType.DMA for _ in range(NB)],
        compiler_params=pltpu.CompilerParams(use_tc_tiling_on_sc=False),
    )
    def agg_kernel(y_hbm, src_hbm, dst_hbm, zero_hbm, out_hbm,
                   src_v, dst_v, stage_v, acc_sh, *ring):
        rows = ring[:NB]
        sems = ring[NB:]
        c = lax.axis_index("c")
        s = lax.axis_index("s")
        wid = s * NC + c
        pltpu.sync_copy(src_hbm.at[wid], src_v)
        pltpu.sync_copy(dst_hbm.at[wid], dst_v)
        r0 = s * zrows
        pltpu.sync_copy(zero_hbm.at[pl.ds(r0, zrows)], stage_v)
        pltpu.sync_copy(stage_v, acc_sh.at[pl.ds(r0, zrows)])
        plsc.subcore_barrier()

        for b in range(NB):
            pltpu.make_async_copy(y_hbm.at[src_v.at[b]], rows[b], sems[b]).start()

        def body(g, carry):
            for b in range(NB):
                j = g * NB + b
                pltpu.make_async_copy(y_hbm.at[src_v.at[j]], rows[b], sems[b]).wait()
                pltpu.sync_copy(rows[b], acc_sh.at[dst_v.at[j]], add=True)
                jn = j + NB

                @pl.when(jn < nch)
                def _():
                    pltpu.make_async_copy(
                        y_hbm.at[src_v.at[jn]], rows[b], sems[b]).start()
            return carry

        lax.fori_loop(0, nch // NB, body, 0)
        plsc.subcore_barrier()
        pltpu.sync_copy(acc_sh.at[pl.ds(r0, zrows)], stage_v)
        pltpu.sync_copy(stage_v, out_hbm.at[c, pl.ds(r0, zrows)])

    return agg_kernel(y, src3, dst3, zeros_a)


def _tc_dense1(x, w1p, degp, n, hp):
    def body(x_ref, w_ref, deg_ref, xw_ref, y_ref, dinv_ref):
        xw = jnp.dot(x_ref[...], w_ref[...], preferred_element_type=jnp.float32)
        deg = deg_ref[0, :n, :1] + deg_ref[1, :n, :1] + 1.0
        dinv = lax.rsqrt(deg)
        xw_ref[...] = xw
        y_ref[...] = xw * dinv
        dinv_ref[...] = dinv

    f32 = jnp.float32
    return pl.pallas_call(
        body,
        out_shape=(
            jax.ShapeDtypeStruct((n, hp), f32),
            jax.ShapeDtypeStruct((n, hp), f32),
            jax.ShapeDtypeStruct((n, 1), f32),
        ),
    )(x, w1p, degp)


def _tc_mid(zp, xw1, dinv, b1p, w2p, n, hp):
    def body(z_ref, xw_ref, dinv_ref, b_ref, w_ref, xw2_ref, y2_ref):
        dinv_v = dinv_ref[...]
        z = z_ref[0, :n, :] + z_ref[1, :n, :]
        h = jnp.maximum(z * dinv_v + xw_ref[...] * (dinv_v * dinv_v) + b_ref[...], 0.0)
        xw2 = jnp.dot(h, w_ref[...], preferred_element_type=jnp.float32)
        xw2_ref[...] = xw2
        y2_ref[...] = xw2 * dinv_v

    f32 = jnp.float32
    return pl.pallas_call(
        body,
        out_shape=(
            jax.ShapeDtypeStruct((n, hp), f32),
            jax.ShapeDtypeStruct((n, hp), f32),
        ),
    )(zp, xw1, dinv, b1p, w2p)


def _tc_final(zp, xw2, dinv, b2p, batch2, w3p, b3p, n, g, o):
    def body(z_ref, xw_ref, dinv_ref, b_ref, bt_ref, w3_ref, b3_ref, out_ref):
        dinv_v = dinv_ref[...]
        z = z_ref[0, :n, :] + z_ref[1, :n, :]
        h = jnp.maximum(z * dinv_v + xw_ref[...] * (dinv_v * dinv_v) + b_ref[...], 0.0)
        gid = lax.broadcasted_iota(jnp.int32, (n, g), 1)
        m = (bt_ref[...] == gid).astype(jnp.float32)
        sums = lax.dot_general(m, h, (((0,), (0,)), ((), ())),
                               preferred_element_type=jnp.float32)
        cnt = jnp.sum(m, axis=0)
        mean = sums / jnp.maximum(cnt, 1.0)[:, None]
        out_ref[...] = jnp.dot(mean, w3_ref[...],
                               preferred_element_type=jnp.float32) + b3_ref[...]

    return pl.pallas_call(
        body,
        out_shape=jax.ShapeDtypeStruct((g, o), jnp.float32),
    )(zp, xw2, dinv, b2p, batch2, w3p, b3p)


def kernel(x, edge_index, batch, W1, b1, W2, b2, W3, b3):
    n, d = x.shape
    e = edge_index.shape[1]
    h = W1.shape[1]
    o = W3.shape[1]
    g = 64
    hp = h
    n_pad = _cdiv(n + 64, NS * 8) * (NS * 8)

    epw = _cdiv(e, NW)
    nch = _cdiv(_cdiv(epw, CHUNK), NB) * NB
    epw_pad = nch * CHUNK
    pad_e = NW * epw_pad - e

    src = edge_index[0].reshape(-1)
    dst = edge_index[1].reshape(-1)
    ar = jnp.arange(pad_e, dtype=jnp.int32)
    src_pad = (ar * 37) % n
    dst_pad = n + (ar % 64)
    src3 = jnp.concatenate([src, src_pad]).reshape(NW, nch, CHUNK)
    dst3 = jnp.concatenate([dst, dst_pad]).reshape(NW, nch, CHUNK)

    f32 = jnp.float32
    w1p = jnp.pad(W1, ((0, 0), (0, hp - h)))
    w2p = jnp.pad(W2, ((0, hp - h), (0, hp - h)))
    w3p = jnp.pad(W3, ((0, hp - h), (0, 0)))
    b1p = jnp.pad(b1, (0, hp - h))[None, :]
    b2p = jnp.pad(b2, (0, hp - h))[None, :]
    b3p = b3[None, :]
    batch2 = batch[:, None]

    ones_c = jnp.ones((CHUNK, DW), f32)
    zeros_d = jnp.zeros((n_pad, DW), f32)
    zeros_a = jnp.zeros((n_pad, hp), f32)

    degp = _sc_degree(dst3, ones_c, zeros_d, n_pad=n_pad, nch=nch)
    xw1, y1, dinv = _tc_dense1(x, w1p, degp, n, hp)
    z1 = _sc_aggregate(y1, src3, dst3, zeros_a, n_pad=n_pad, nch=nch, hp=hp)
    xw2, y2 = _tc_mid(z1, xw1, dinv, b1p, w2p, n, hp)
    z2 = _sc_aggregate(y2, src3, dst3, zeros_a, n_pad=n_pad, nch=nch, hp=hp)
    return _tc_final(z2, xw2, dinv, b2p, batch2, w3p, b3p, n, g, o)


# speedup vs baseline: 1.0073x; 1.0073x over previous
import functools

import jax
import jax.numpy as jnp
from jax import lax
from jax.experimental import pallas as pl
from jax.experimental.pallas import tpu as pltpu
from jax.experimental.pallas import tpu_sc as plsc

NC = 2
NS = 16
NW = NC * NS
CHUNK = 128


def _cdiv(a, b):
    return (a + b - 1) // b


DW = 8


@functools.partial(jax.jit, static_argnames=("n_pad", "nch"))
def _sc_degree(dst3, ones_c, zeros_d, *, n_pad, nch):
    zrows = n_pad // NS
    mesh = plsc.VectorSubcoreMesh(core_axis_name="c", subcore_axis_name="s")

    @functools.partial(
        pl.kernel,
        out_type=jax.ShapeDtypeStruct((NC, n_pad, DW), jnp.float32),
        mesh=mesh,
        scratch_types=[
            pltpu.VMEM((nch, CHUNK), jnp.int32),
            pltpu.VMEM((CHUNK, DW), jnp.float32),
            pltpu.VMEM((zrows, DW), jnp.float32),
            pltpu.VMEM_SHARED((n_pad, DW), jnp.float32),
        ],
        compiler_params=pltpu.CompilerParams(use_tc_tiling_on_sc=False),
    )
    def deg_kernel(dst_hbm, ones_hbm, zero_hbm, out_hbm,
                   dst_v, ones_v, stage_v, acc_sh):
        c = lax.axis_index("c")
        s = lax.axis_index("s")
        wid = s * NC + c
        pltpu.sync_copy(dst_hbm.at[wid], dst_v)
        pltpu.sync_copy(ones_hbm, ones_v)
        r0 = s * zrows
        pltpu.sync_copy(zero_hbm.at[pl.ds(r0, zrows)], stage_v)
        pltpu.sync_copy(stage_v, acc_sh.at[pl.ds(r0, zrows)])
        plsc.subcore_barrier()

        def body(j, carry):
            pltpu.sync_copy(ones_v, acc_sh.at[dst_v.at[j]], add=True)
            return carry

        lax.fori_loop(0, nch, body, 0)
        plsc.subcore_barrier()
        pltpu.sync_copy(acc_sh.at[pl.ds(r0, zrows)], stage_v)
        pltpu.sync_copy(stage_v, out_hbm.at[c, pl.ds(r0, zrows)])

    return deg_kernel(dst3, ones_c, zeros_d)


NB = 4


@functools.partial(jax.jit, static_argnames=("n_pad", "nch", "hp"))
def _sc_aggregate(y, src3, dst3, zeros_a, *, n_pad, nch, hp):
    zrows = n_pad // NS
    mesh = plsc.VectorSubcoreMesh(core_axis_name="c", subcore_axis_name="s")

    @functools.partial(
        pl.kernel,
        out_type=jax.ShapeDtypeStruct((NC, n_pad, hp), jnp.float32),
        mesh=mesh,
        scratch_types=[
            pltpu.VMEM((nch, CHUNK), jnp.int32),
            pltpu.VMEM((nch, CHUNK), jnp.int32),
            pltpu.VMEM((zrows, hp), jnp.float32),
            pltpu.VMEM_SHARED((n_pad, hp), jnp.float32),
        ]
        + [pltpu.VMEM((CHUNK, hp), jnp.float32) for _ in range(NB)]
        + [pltpu.SemaphoreType.DMA for _ in range(NB)],
        compiler_params=pltpu.CompilerParams(use_tc_tiling_on_sc=False),
    )
    def agg_kernel(y_hbm, src_hbm, dst_hbm, zero_hbm, out_hbm,
                   src_v, dst_v, stage_v, acc_sh, *ring):
        rows = ring[:NB]
        sems = ring[NB:]
        c = lax.axis_index("c")
        s = lax.axis_index("s")
        wid = s * NC + c
        pltpu.sync_copy(src_hbm.at[wid], src_v)
        pltpu.sync_copy(dst_hbm.at[wid], dst_v)
        r0 = s * zrows
        pltpu.sync_copy(zero_hbm.at[pl.ds(r0, zrows)], stage_v)
        pltpu.sync_copy(stage_v, acc_sh.at[pl.ds(r0, zrows)])
        plsc.subcore_barrier()

        for b in range(NB):
            pltpu.make_async_copy(y_hbm.at[src_v.at[b]], rows[b], sems[b]).start()

        def body(g, carry):
            for b in range(NB):
                j = g * NB + b
                pltpu.make_async_copy(y_hbm.at[src_v.at[j]], rows[b], sems[b]).wait()
                pltpu.sync_copy(rows[b], acc_sh.at[dst_v.at[j]], add=True)
                jn = j + NB

                @pl.when(jn < nch)
                def _():
                    pltpu.make_async_copy(
                        y_hbm.at[src_v.at[jn]], rows[b], sems[b]).start()
            return carry

        lax.fori_loop(0, nch // NB, body, 0)
        plsc.subcore_barrier()
        pltpu.sync_copy(acc_sh.at[pl.ds(r0, zrows)], stage_v)
        pltpu.sync_copy(stage_v, out_hbm.at[c, pl.ds(r0, zrows)])

    return agg_kernel(y, src3, dst3, zeros_a)


def _tc_dense1(x, w1p, degp, n, hp):
    def body(x_ref, w_ref, deg_ref, xw_ref, y_ref, dinv_ref):
        xw = jnp.dot(x_ref[...], w_ref[...], preferred_element_type=jnp.float32)
        deg = deg_ref[0, :n, :1] + deg_ref[1, :n, :1] + 1.0
        dinv = lax.rsqrt(deg)
        xw_ref[...] = xw
        y_ref[...] = xw * dinv
        dinv_ref[...] = dinv

    f32 = jnp.float32
    return pl.pallas_call(
        body,
        out_shape=(
            jax.ShapeDtypeStruct((n, hp), f32),
            jax.ShapeDtypeStruct((n, hp), f32),
            jax.ShapeDtypeStruct((n, 1), f32),
        ),
    )(x, w1p, degp)


def _tc_mid(zp, xw1, dinv, b1p, w2p, n, hp):
    def body(z_ref, xw_ref, dinv_ref, b_ref, w_ref, xw2_ref, y2_ref):
        dinv_v = dinv_ref[...]
        z = z_ref[0, :n, :] + z_ref[1, :n, :]
        h = jnp.maximum(z * dinv_v + xw_ref[...] * (dinv_v * dinv_v) + b_ref[...], 0.0)
        xw2 = jnp.dot(h, w_ref[...], preferred_element_type=jnp.float32)
        xw2_ref[...] = xw2
        y2_ref[...] = xw2 * dinv_v

    f32 = jnp.float32
    return pl.pallas_call(
        body,
        out_shape=(
            jax.ShapeDtypeStruct((n, hp), f32),
            jax.ShapeDtypeStruct((n, hp), f32),
        ),
    )(zp, xw1, dinv, b1p, w2p)


def _tc_final(zp, xw2, dinv, b2p, batch2, w3p, b3p, n, g, o):
    def body(z_ref, xw_ref, dinv_ref, b_ref, bt_ref, w3_ref, b3_ref, out_ref):
        dinv_v = dinv_ref[...]
        z = z_ref[0, :n, :] + z_ref[1, :n, :]
        h = jnp.maximum(z * dinv_v + xw_ref[...] * (dinv_v * dinv_v) + b_ref[...], 0.0)
        gid = lax.broadcasted_iota(jnp.int32, (n, g), 1)
        m = (bt_ref[...] == gid).astype(jnp.float32)
        sums = lax.dot_general(m, h, (((0,), (0,)), ((), ())),
                               preferred_element_type=jnp.float32)
        cnt = jnp.sum(m, axis=0)
        mean = sums / jnp.maximum(cnt, 1.0)[:, None]
        out_ref[...] = jnp.dot(mean, w3_ref[...],
                               preferred_element_type=jnp.float32) + b3_ref[...]

    return pl.pallas_call(
        body,
        out_shape=jax.ShapeDtypeStruct((g, o), jnp.float32),
    )(zp, xw2, dinv, b2p, batch2, w3p, b3p)


def kernel(x, edge_index, batch, W1, b1, W2, b2, W3, b3):
    n, d = x.shape
    e = edge_index.shape[1]
    h = W1.shape[1]
    o = W3.shape[1]
    g = 64
    hp = 32
    n_pad = _cdiv(n + 64, NS * 8) * (NS * 8)

    epw = _cdiv(e, NW)
    nch = _cdiv(_cdiv(epw, CHUNK), NB) * NB
    epw_pad = nch * CHUNK
    pad_e = NW * epw_pad - e

    src = edge_index[0].reshape(-1)
    dst = edge_index[1].reshape(-1)
    ar = jnp.arange(pad_e, dtype=jnp.int32)
    src_pad = (ar * 37) % n
    dst_pad = n + (ar % 64)
    src3 = jnp.concatenate([src, src_pad]).reshape(NW, nch, CHUNK)
    dst3 = jnp.concatenate([dst, dst_pad]).reshape(NW, nch, CHUNK)

    f32 = jnp.float32
    w1p = jnp.pad(W1, ((0, 0), (0, hp - h)))
    w2p = jnp.pad(W2, ((0, hp - h), (0, hp - h)))
    w3p = jnp.pad(W3, ((0, hp - h), (0, 0)))
    b1p = jnp.pad(b1, (0, hp - h))[None, :]
    b2p = jnp.pad(b2, (0, hp - h))[None, :]
    b3p = b3[None, :]
    batch2 = batch[:, None]

    ones_c = jnp.ones((CHUNK, DW), f32)
    zeros_d = jnp.zeros((n_pad, DW), f32)
    zeros_a = jnp.zeros((n_pad, hp), f32)

    degp = _sc_degree(dst3, ones_c, zeros_d, n_pad=n_pad, nch=nch)
    xw1, y1, dinv = _tc_dense1(x, w1p, degp, n, hp)
    z1 = _sc_aggregate(y1, src3, dst3, zeros_a, n_pad=n_pad, nch=nch, hp=hp)
    xw2, y2 = _tc_mid(z1, xw1, dinv, b1p, w2p, n, hp)
    z2 = _sc_aggregate(y2, src3, dst3, zeros_a, n_pad=n_pad, nch=nch, hp=hp)
    return _tc_final(z2, xw2, dinv, b2p, batch2, w3p, b3p, n, g, o)


# speedup vs baseline: 1.0325x; 1.0249x over previous
import functools

import jax
import jax.numpy as jnp
from jax import lax
from jax.experimental import pallas as pl
from jax.experimental.pallas import tpu as pltpu
from jax.experimental.pallas import tpu_sc as plsc

NC = 2
NS = 16
NW = NC * NS
CHUNK = 128


def _cdiv(a, b):
    return (a + b - 1) // b


DW = 8


@functools.partial(jax.jit, static_argnames=("n_pad", "nch"))
def _sc_degree(dst3, ones_c, zeros_d, *, n_pad, nch):
    zrows = n_pad // NS
    mesh = plsc.VectorSubcoreMesh(core_axis_name="c", subcore_axis_name="s")

    @functools.partial(
        pl.kernel,
        out_type=jax.ShapeDtypeStruct((NC, n_pad, DW), jnp.float32),
        mesh=mesh,
        scratch_types=[
            pltpu.VMEM((nch, CHUNK), jnp.int32),
            pltpu.VMEM((CHUNK, DW), jnp.float32),
            pltpu.VMEM((zrows, DW), jnp.float32),
            pltpu.VMEM_SHARED((n_pad, DW), jnp.float32),
            pltpu.SemaphoreType.DMA,
        ],
        compiler_params=pltpu.CompilerParams(use_tc_tiling_on_sc=False),
    )
    def deg_kernel(dst_hbm, ones_hbm, zero_hbm, out_hbm,
                   dst_v, ones_v, stage_v, acc_sh, sem):
        c = lax.axis_index("c")
        s = lax.axis_index("s")
        wid = s * NC + c
        pltpu.sync_copy(dst_hbm.at[wid], dst_v)
        pltpu.sync_copy(ones_hbm, ones_v)
        r0 = s * zrows
        pltpu.sync_copy(zero_hbm.at[pl.ds(r0, zrows)], stage_v)
        pltpu.sync_copy(stage_v, acc_sh.at[pl.ds(r0, zrows)])
        plsc.subcore_barrier()

        def fire(j, carry):
            pltpu.make_async_copy(ones_v, acc_sh.at[dst_v.at[j]], sem).start(
                add=True)
            return carry

        def drain(j, carry):
            pltpu.make_async_copy(ones_v, acc_sh.at[dst_v.at[j]], sem).wait()
            return carry

        lax.fori_loop(0, nch, fire, 0)
        lax.fori_loop(0, nch, drain, 0)
        plsc.subcore_barrier()
        pltpu.sync_copy(acc_sh.at[pl.ds(r0, zrows)], stage_v)
        pltpu.sync_copy(stage_v, out_hbm.at[c, pl.ds(r0, zrows)])

    return deg_kernel(dst3, ones_c, zeros_d)


NB = 4


@functools.partial(jax.jit, static_argnames=("n_pad", "nch", "hp"))
def _sc_aggregate(y, src3, dst3, zeros_a, *, n_pad, nch, hp):
    zrows = n_pad // NS
    mesh = plsc.VectorSubcoreMesh(core_axis_name="c", subcore_axis_name="s")

    @functools.partial(
        pl.kernel,
        out_type=jax.ShapeDtypeStruct((NC, n_pad, hp), jnp.float32),
        mesh=mesh,
        scratch_types=[
            pltpu.VMEM((nch, CHUNK), jnp.int32),
            pltpu.VMEM((nch, CHUNK), jnp.int32),
            pltpu.VMEM((zrows, hp), jnp.float32),
            pltpu.VMEM_SHARED((n_pad, hp), jnp.float32),
        ]
        + [pltpu.VMEM((CHUNK, hp), jnp.float32) for _ in range(NB)]
        + [pltpu.SemaphoreType.DMA for _ in range(NB)],
        compiler_params=pltpu.CompilerParams(use_tc_tiling_on_sc=False),
    )
    def agg_kernel(y_hbm, src_hbm, dst_hbm, zero_hbm, out_hbm,
                   src_v, dst_v, stage_v, acc_sh, *ring):
        rows = ring[:NB]
        sems = ring[NB:]
        c = lax.axis_index("c")
        s = lax.axis_index("s")
        wid = s * NC + c
        pltpu.sync_copy(src_hbm.at[wid], src_v)
        pltpu.sync_copy(dst_hbm.at[wid], dst_v)
        r0 = s * zrows
        pltpu.sync_copy(zero_hbm.at[pl.ds(r0, zrows)], stage_v)
        pltpu.sync_copy(stage_v, acc_sh.at[pl.ds(r0, zrows)])
        plsc.subcore_barrier()

        for b in range(NB):
            pltpu.make_async_copy(y_hbm.at[src_v.at[b]], rows[b], sems[b]).start()

        def body(g, carry):
            for b in range(NB):
                j = g * NB + b
                pltpu.make_async_copy(y_hbm.at[src_v.at[j]], rows[b], sems[b]).wait()
                pltpu.sync_copy(rows[b], acc_sh.at[dst_v.at[j]], add=True)
                jn = j + NB

                @pl.when(jn < nch)
                def _():
                    pltpu.make_async_copy(
                        y_hbm.at[src_v.at[jn]], rows[b], sems[b]).start()
            return carry

        lax.fori_loop(0, nch // NB, body, 0)
        plsc.subcore_barrier()
        pltpu.sync_copy(acc_sh.at[pl.ds(r0, zrows)], stage_v)
        pltpu.sync_copy(stage_v, out_hbm.at[c, pl.ds(r0, zrows)])

    return agg_kernel(y, src3, dst3, zeros_a)


def _tc_dense1(x, w1p, degp, n, hp):
    def body(x_ref, w_ref, deg_ref, xw_ref, y_ref, dinv_ref):
        xw = jnp.dot(x_ref[...], w_ref[...], preferred_element_type=jnp.float32)
        deg = deg_ref[0, :n, :1] + deg_ref[1, :n, :1] + 1.0
        dinv = lax.rsqrt(deg)
        xw_ref[...] = xw
        y_ref[...] = xw * dinv
        dinv_ref[...] = dinv

    f32 = jnp.float32
    return pl.pallas_call(
        body,
        out_shape=(
            jax.ShapeDtypeStruct((n, hp), f32),
            jax.ShapeDtypeStruct((n, hp), f32),
            jax.ShapeDtypeStruct((n, 1), f32),
        ),
    )(x, w1p, degp)


def _tc_mid(zp, xw1, dinv, b1p, w2p, n, hp):
    def body(z_ref, xw_ref, dinv_ref, b_ref, w_ref, xw2_ref, y2_ref):
        dinv_v = dinv_ref[...]
        z = z_ref[0, :n, :] + z_ref[1, :n, :]
        h = jnp.maximum(z * dinv_v + xw_ref[...] * (dinv_v * dinv_v) + b_ref[...], 0.0)
        xw2 = jnp.dot(h, w_ref[...], preferred_element_type=jnp.float32)
        xw2_ref[...] = xw2
        y2_ref[...] = xw2 * dinv_v

    f32 = jnp.float32
    return pl.pallas_call(
        body,
        out_shape=(
            jax.ShapeDtypeStruct((n, hp), f32),
            jax.ShapeDtypeStruct((n, hp), f32),
        ),
    )(zp, xw1, dinv, b1p, w2p)


def _tc_final(zp, xw2, dinv, b2p, batch2, w3p, b3p, n, g, o):
    def body(z_ref, xw_ref, dinv_ref, b_ref, bt_ref, w3_ref, b3_ref, out_ref):
        dinv_v = dinv_ref[...]
        z = z_ref[0, :n, :] + z_ref[1, :n, :]
        h = jnp.maximum(z * dinv_v + xw_ref[...] * (dinv_v * dinv_v) + b_ref[...], 0.0)
        gid = lax.broadcasted_iota(jnp.int32, (n, g), 1)
        m = (bt_ref[...] == gid).astype(jnp.float32)
        sums = lax.dot_general(m, h, (((0,), (0,)), ((), ())),
                               preferred_element_type=jnp.float32)
        cnt = jnp.sum(m, axis=0)
        mean = sums / jnp.maximum(cnt, 1.0)[:, None]
        out_ref[...] = jnp.dot(mean, w3_ref[...],
                               preferred_element_type=jnp.float32) + b3_ref[...]

    return pl.pallas_call(
        body,
        out_shape=jax.ShapeDtypeStruct((g, o), jnp.float32),
    )(zp, xw2, dinv, b2p, batch2, w3p, b3p)


def kernel(x, edge_index, batch, W1, b1, W2, b2, W3, b3):
    n, d = x.shape
    e = edge_index.shape[1]
    h = W1.shape[1]
    o = W3.shape[1]
    g = 64
    hp = 32
    n_pad = _cdiv(n + 64, NS * 8) * (NS * 8)

    epw = _cdiv(e, NW)
    nch = _cdiv(_cdiv(epw, CHUNK), NB) * NB
    epw_pad = nch * CHUNK
    pad_e = NW * epw_pad - e

    src = edge_index[0].reshape(-1)
    dst = edge_index[1].reshape(-1)
    ar = jnp.arange(pad_e, dtype=jnp.int32)
    src_pad = (ar * 37) % n
    dst_pad = n + (ar % 64)
    src3 = jnp.concatenate([src, src_pad]).reshape(NW, nch, CHUNK)
    dst3 = jnp.concatenate([dst, dst_pad]).reshape(NW, nch, CHUNK)

    f32 = jnp.float32
    w1p = jnp.pad(W1, ((0, 0), (0, hp - h)))
    w2p = jnp.pad(W2, ((0, hp - h), (0, hp - h)))
    w3p = jnp.pad(W3, ((0, hp - h), (0, 0)))
    b1p = jnp.pad(b1, (0, hp - h))[None, :]
    b2p = jnp.pad(b2, (0, hp - h))[None, :]
    b3p = b3[None, :]
    batch2 = batch[:, None]

    ones_c = jnp.ones((CHUNK, DW), f32)
    zeros_d = jnp.zeros((n_pad, DW), f32)
    zeros_a = jnp.zeros((n_pad, hp), f32)

    degp = _sc_degree(dst3, ones_c, zeros_d, n_pad=n_pad, nch=nch)
    xw1, y1, dinv = _tc_dense1(x, w1p, degp, n, hp)
    z1 = _sc_aggregate(y1, src3, dst3, zeros_a, n_pad=n_pad, nch=nch, hp=hp)
    xw2, y2 = _tc_mid(z1, xw1, dinv, b1p, w2p, n, hp)
    z2 = _sc_aggregate(y2, src3, dst3, zeros_a, n_pad=n_pad, nch=nch, hp=hp)
    return _tc_final(z2, xw2, dinv, b2p, batch2, w3p, b3p, n, g, o)


# speedup vs baseline: 1.0700x; 1.0363x over previous
import functools

import jax
import jax.numpy as jnp
from jax import lax
from jax.experimental import pallas as pl
from jax.experimental.pallas import tpu as pltpu
from jax.experimental.pallas import tpu_sc as plsc

NC = 2
NS = 16
NW = NC * NS
CHUNK = 128


def _cdiv(a, b):
    return (a + b - 1) // b


DW = 8


@functools.partial(jax.jit, static_argnames=("n_pad", "nch"))
def _sc_degree(dst3, ones_c, zeros_d, *, n_pad, nch):
    zrows = n_pad // NS
    mesh = plsc.VectorSubcoreMesh(core_axis_name="c", subcore_axis_name="s")

    @functools.partial(
        pl.kernel,
        out_type=jax.ShapeDtypeStruct((NC, n_pad, DW), jnp.float32),
        mesh=mesh,
        scratch_types=[
            pltpu.VMEM((nch, CHUNK), jnp.int32),
            pltpu.VMEM((CHUNK, DW), jnp.float32),
            pltpu.VMEM((zrows, DW), jnp.float32),
            pltpu.VMEM_SHARED((n_pad, DW), jnp.float32),
            pltpu.SemaphoreType.DMA,
        ],
        compiler_params=pltpu.CompilerParams(use_tc_tiling_on_sc=False),
    )
    def deg_kernel(dst_hbm, ones_hbm, zero_hbm, out_hbm,
                   dst_v, ones_v, stage_v, acc_sh, sem):
        c = lax.axis_index("c")
        s = lax.axis_index("s")
        wid = s * NC + c
        pltpu.sync_copy(dst_hbm.at[wid], dst_v)
        pltpu.sync_copy(ones_hbm, ones_v)
        r0 = s * zrows
        pltpu.sync_copy(zero_hbm.at[pl.ds(r0, zrows)], stage_v)
        pltpu.sync_copy(stage_v, acc_sh.at[pl.ds(r0, zrows)])
        plsc.subcore_barrier()

        def fire(j, carry):
            pltpu.make_async_copy(ones_v, acc_sh.at[dst_v.at[j]], sem).start(
                add=True)
            return carry

        def drain(j, carry):
            pltpu.make_async_copy(ones_v, acc_sh.at[dst_v.at[j]], sem).wait()
            return carry

        lax.fori_loop(0, nch, fire, 0)
        lax.fori_loop(0, nch, drain, 0)
        plsc.subcore_barrier()
        pltpu.sync_copy(acc_sh.at[pl.ds(r0, zrows)], stage_v)
        pltpu.sync_copy(stage_v, out_hbm.at[c, pl.ds(r0, zrows)])

    return deg_kernel(dst3, ones_c, zeros_d)


NB = 8


@functools.partial(jax.jit, static_argnames=("n_pad", "nch", "hp"))
def _sc_aggregate(y, src3, dst3, zeros_a, *, n_pad, nch, hp):
    zrows = n_pad // NS
    mesh = plsc.VectorSubcoreMesh(core_axis_name="c", subcore_axis_name="s")

    @functools.partial(
        pl.kernel,
        out_type=jax.ShapeDtypeStruct((NC, n_pad, hp), jnp.float32),
        mesh=mesh,
        scratch_types=[
            pltpu.VMEM((nch, CHUNK), jnp.int32),
            pltpu.VMEM((nch, CHUNK), jnp.int32),
            pltpu.VMEM((zrows, hp), jnp.float32),
            pltpu.VMEM_SHARED((n_pad, hp), jnp.float32),
        ]
        + [pltpu.VMEM((CHUNK, hp), jnp.float32) for _ in range(NB)]
        + [pltpu.SemaphoreType.DMA for _ in range(NB)],
        compiler_params=pltpu.CompilerParams(use_tc_tiling_on_sc=False),
    )
    def agg_kernel(y_hbm, src_hbm, dst_hbm, zero_hbm, out_hbm,
                   src_v, dst_v, stage_v, acc_sh, *ring):
        rows = ring[:NB]
        sems = ring[NB:]
        c = lax.axis_index("c")
        s = lax.axis_index("s")
        wid = s * NC + c
        pltpu.sync_copy(src_hbm.at[wid], src_v)
        pltpu.sync_copy(dst_hbm.at[wid], dst_v)
        r0 = s * zrows
        pltpu.sync_copy(zero_hbm.at[pl.ds(r0, zrows)], stage_v)
        pltpu.sync_copy(stage_v, acc_sh.at[pl.ds(r0, zrows)])
        plsc.subcore_barrier()

        for b in range(NB):
            pltpu.make_async_copy(y_hbm.at[src_v.at[b]], rows[b], sems[b]).start()

        def body(g, carry):
            for b in range(NB):
                j = g * NB + b
                pltpu.make_async_copy(y_hbm.at[src_v.at[j]], rows[b], sems[b]).wait()
                pltpu.sync_copy(rows[b], acc_sh.at[dst_v.at[j]], add=True)
                jn = j + NB

                @pl.when(jn < nch)
                def _():
                    pltpu.make_async_copy(
                        y_hbm.at[src_v.at[jn]], rows[b], sems[b]).start()
            return carry

        lax.fori_loop(0, nch // NB, body, 0)
        plsc.subcore_barrier()
        pltpu.sync_copy(acc_sh.at[pl.ds(r0, zrows)], stage_v)
        pltpu.sync_copy(stage_v, out_hbm.at[c, pl.ds(r0, zrows)])

    return agg_kernel(y, src3, dst3, zeros_a)


def _tc_dense1(x, w1p, degp, n, hp):
    def body(x_ref, w_ref, deg_ref, xw_ref, y_ref, dinv_ref):
        xw = jnp.dot(x_ref[...], w_ref[...], preferred_element_type=jnp.float32)
        deg = deg_ref[0, :n, :1] + deg_ref[1, :n, :1] + 1.0
        dinv = lax.rsqrt(deg)
        xw_ref[...] = xw
        y_ref[...] = xw * dinv
        dinv_ref[...] = dinv

    f32 = jnp.float32
    return pl.pallas_call(
        body,
        out_shape=(
            jax.ShapeDtypeStruct((n, hp), f32),
            jax.ShapeDtypeStruct((n, hp), f32),
            jax.ShapeDtypeStruct((n, 1), f32),
        ),
    )(x, w1p, degp)


def _tc_mid(zp, xw1, dinv, b1p, w2p, n, hp):
    def body(z_ref, xw_ref, dinv_ref, b_ref, w_ref, xw2_ref, y2_ref):
        dinv_v = dinv_ref[...]
        z = z_ref[0, :n, :] + z_ref[1, :n, :]
        h = jnp.maximum(z * dinv_v + xw_ref[...] * (dinv_v * dinv_v) + b_ref[...], 0.0)
        xw2 = jnp.dot(h, w_ref[...], preferred_element_type=jnp.float32)
        xw2_ref[...] = xw2
        y2_ref[...] = xw2 * dinv_v

    f32 = jnp.float32
    return pl.pallas_call(
        body,
        out_shape=(
            jax.ShapeDtypeStruct((n, hp), f32),
            jax.ShapeDtypeStruct((n, hp), f32),
        ),
    )(zp, xw1, dinv, b1p, w2p)


def _tc_final(zp, xw2, dinv, b2p, batch2, w3p, b3p, n, g, o):
    def body(z_ref, xw_ref, dinv_ref, b_ref, bt_ref, w3_ref, b3_ref, out_ref):
        dinv_v = dinv_ref[...]
        z = z_ref[0, :n, :] + z_ref[1, :n, :]
        h = jnp.maximum(z * dinv_v + xw_ref[...] * (dinv_v * dinv_v) + b_ref[...], 0.0)
        gid = lax.broadcasted_iota(jnp.int32, (n, g), 1)
        m = (bt_ref[...] == gid).astype(jnp.float32)
        sums = lax.dot_general(m, h, (((0,), (0,)), ((), ())),
                               preferred_element_type=jnp.float32)
        cnt = jnp.sum(m, axis=0)
        mean = sums / jnp.maximum(cnt, 1.0)[:, None]
        out_ref[...] = jnp.dot(mean, w3_ref[...],
                               preferred_element_type=jnp.float32) + b3_ref[...]

    return pl.pallas_call(
        body,
        out_shape=jax.ShapeDtypeStruct((g, o), jnp.float32),
    )(zp, xw2, dinv, b2p, batch2, w3p, b3p)


def kernel(x, edge_index, batch, W1, b1, W2, b2, W3, b3):
    n, d = x.shape
    e = edge_index.shape[1]
    h = W1.shape[1]
    o = W3.shape[1]
    g = 64
    hp = 32
    n_pad = _cdiv(n + 64, NS * 8) * (NS * 8)

    epw = _cdiv(e, NW)
    nch = _cdiv(_cdiv(epw, CHUNK), NB) * NB
    epw_pad = nch * CHUNK
    pad_e = NW * epw_pad - e

    src = edge_index[0].reshape(-1)
    dst = edge_index[1].reshape(-1)
    ar = jnp.arange(pad_e, dtype=jnp.int32)
    src_pad = (ar * 37) % n
    dst_pad = n + (ar % 64)
    src3 = jnp.concatenate([src, src_pad]).reshape(NW, nch, CHUNK)
    dst3 = jnp.concatenate([dst, dst_pad]).reshape(NW, nch, CHUNK)

    f32 = jnp.float32
    w1p = jnp.pad(W1, ((0, 0), (0, hp - h)))
    w2p = jnp.pad(W2, ((0, hp - h), (0, hp - h)))
    w3p = jnp.pad(W3, ((0, hp - h), (0, 0)))
    b1p = jnp.pad(b1, (0, hp - h))[None, :]
    b2p = jnp.pad(b2, (0, hp - h))[None, :]
    b3p = b3[None, :]
    batch2 = batch[:, None]

    ones_c = jnp.ones((CHUNK, DW), f32)
    zeros_d = jnp.zeros((n_pad, DW), f32)
    zeros_a = jnp.zeros((n_pad, hp), f32)

    degp = _sc_degree(dst3, ones_c, zeros_d, n_pad=n_pad, nch=nch)
    xw1, y1, dinv = _tc_dense1(x, w1p, degp, n, hp)
    z1 = _sc_aggregate(y1, src3, dst3, zeros_a, n_pad=n_pad, nch=nch, hp=hp)
    xw2, y2 = _tc_mid(z1, xw1, dinv, b1p, w2p, n, hp)
    z2 = _sc_aggregate(y2, src3, dst3, zeros_a, n_pad=n_pad, nch=nch, hp=hp)
    return _tc_final(z2, xw2, dinv, b2p, batch2, w3p, b3p, n, g, o)
